# Initial kernel scaffold; baseline (speedup 1.0000x reference)
import jax
import jax.numpy as jnp
from jax.experimental import pallas as pl


def kernel(flat, cu_seqlens, match_ends, W1, w2):
    raise NotImplementedError("write your pallas kernel here")


# speedup vs baseline: 2.5197x; 2.5197x over previous
import functools

import jax
import jax.numpy as jnp
from jax import lax
from jax.experimental import pallas as pl
from jax.experimental.pallas import tpu as pltpu
from jax.experimental.pallas import tpu_sc as plsc

_NC = 2
_NS = 16
_NW = _NC * _NS
_L = 16


def _sc_gather(me3, flat):
    nw, chunks, c = me3.shape
    _, d = flat.shape
    m = nw * chunks * c

    mesh = plsc.VectorSubcoreMesh(
        core_axis_name="c", subcore_axis_name="s",
        num_cores=_NC, num_subcores=_NS)

    @functools.partial(
        pl.kernel, mesh=mesh,
        out_type=jax.ShapeDtypeStruct((m, d), jnp.float32),
        scratch_types=[
            pltpu.VMEM((chunks, c), jnp.int32),
            pltpu.VMEM((c, d), jnp.float32),
            pltpu.SemaphoreType.DMA,
        ],
    )
    def gather_k(me_hbm, flat_hbm, out_hbm, idx_v, rows_v, sem):
        wid = lax.axis_index("s") * _NC + lax.axis_index("c")
        pltpu.sync_copy(me_hbm.at[wid], idx_v)
        base = wid * (chunks * c)
        for j in range(chunks):
            pltpu.async_copy(flat_hbm.at[idx_v.at[j]], rows_v, sem).wait()
            pltpu.sync_copy(rows_v, out_hbm.at[pl.ds(base + j * c, c)])

    return gather_k(me3, flat)


def _tc_mixer(gathered, w1b, w2c, bm=1024):
    m, d = gathered.shape
    _, h = w1b.shape

    def body(x_ref, w1_ref, w2_ref, o_ref):
        xb = x_ref[...].astype(jnp.bfloat16)
        acts = jnp.dot(xb, w1_ref[...], preferred_element_type=jnp.float32)
        acts = jax.nn.gelu(acts)
        o_ref[...] = jnp.dot(acts, w2_ref[...],
                             preferred_element_type=jnp.float32)

    return pl.pallas_call(
        body,
        grid=(m // bm,),
        in_specs=[
            pl.BlockSpec((bm, d), lambda i: (i, 0)),
            pl.BlockSpec((d, h), lambda i: (0, 0)),
            pl.BlockSpec((h, 1), lambda i: (0, 0)),
        ],
        out_specs=pl.BlockSpec((bm, 1), lambda i: (i, 0)),
        out_shape=jax.ShapeDtypeStruct((m, 1), jnp.float32),
    )(gathered, w1b, w2c)


def _sc_scatter(preds3, me3, cu_pad, b, max_seqlen):
    ns, rows, c = me3.shape
    out_len = b * max_seqlen
    per_tile = out_len // ns
    nb = cu_pad.shape[0]

    mesh = plsc.VectorSubcoreMesh(
        core_axis_name="c", subcore_axis_name="s",
        num_cores=1, num_subcores=_NS)

    @functools.partial(
        pl.kernel, mesh=mesh,
        out_type=jax.ShapeDtypeStruct((out_len,), jnp.float32),
        scratch_types=[
            pltpu.VMEM((nb,), jnp.int32),
            pltpu.VMEM((rows, c), jnp.float32),
            pltpu.VMEM((rows, c), jnp.int32),
            pltpu.VMEM((rows, c), jnp.int32),
            pltpu.VMEM((per_tile,), jnp.float32),
            pltpu.SemaphoreType.DMA,
        ],
    )
    def scatter_k(preds_hbm, me_hbm, cu_hbm, out_hbm,
                  cu_v, pred_v, me_v, oidx_v, zbuf, sem):
        sid = lax.axis_index("s")
        pltpu.sync_copy(cu_hbm, cu_v)
        pltpu.sync_copy(preds_hbm.at[sid], pred_v)
        pltpu.sync_copy(me_hbm.at[sid], me_v)

        def zero_body(i, _):
            zbuf[pl.ds(i * _L, _L)] = jnp.zeros((_L,), jnp.float32)
            return 0
        lax.fori_loop(0, per_tile // _L, zero_body, 0)
        pltpu.sync_copy(zbuf, out_hbm.at[pl.ds(sid * per_tile, per_tile)])
        plsc.subcore_barrier()

        cuvec = cu_v[pl.ds(0, _L)]
        cu_bcast = [
            cuvec.at[jnp.full((_L,), j, jnp.int32)].get(
                mode="promise_in_bounds")
            for j in range(1, b)
        ]
        for r in range(rows):
            for k in range(c // _L):
                me16 = me_v[r, pl.ds(k * _L, _L)]
                doc16 = jnp.zeros((_L,), jnp.int32)
                for cu_j in cu_bcast:
                    doc16 = doc16 + jnp.where(me16 >= cu_j, 1, 0)
                base16 = cuvec.at[doc16].get(mode="promise_in_bounds")
                oidx_v[r, pl.ds(k * _L, _L)] = (
                    doc16 * max_seqlen + me16 - base16)

        for r in range(rows):
            pltpu.async_copy(pred_v.at[r], out_hbm.at[oidx_v.at[r]],
                             sem).wait()

    return scatter_k(preds3, me3, cu_pad)


def kernel(flat, cu_seqlens, match_ends, W1, w2):
    total_tok, d = flat.shape
    (m,) = match_ends.shape
    b = cu_seqlens.shape[0] - 1
    max_seqlen = 4096

    chunk = 32
    me3 = match_ends.reshape(_NW, m // (_NW * chunk), chunk)
    gathered = _sc_gather(me3, flat)

    preds = _tc_mixer(gathered, W1.astype(jnp.bfloat16),
                      w2.reshape(d, 1).astype(jnp.float32))

    cu_pad = jnp.concatenate(
        [cu_seqlens.astype(jnp.int32),
         jnp.zeros((32 - cu_seqlens.shape[0],), jnp.int32)])
    preds3 = preds.reshape(_NS, m // (_NS * 128), 128)
    me_sc = match_ends.reshape(_NS, m // (_NS * 128), 128)
    out_flat = _sc_scatter(preds3, me_sc, cu_pad, b, max_seqlen)
    return out_flat.reshape(b, max_seqlen)


# speedup vs baseline: 2.5291x; 1.0037x over previous
import functools

import jax
import jax.numpy as jnp
from jax import lax
from jax.experimental import pallas as pl
from jax.experimental.pallas import tpu as pltpu
from jax.experimental.pallas import tpu_sc as plsc

_NC = 2
_NS = 16
_NW = _NC * _NS
_L = 16


def _sc_gather(me3, flat):
    nw, chunks, c = me3.shape
    _, d = flat.shape
    m = nw * chunks * c

    mesh = plsc.VectorSubcoreMesh(
        core_axis_name="c", subcore_axis_name="s",
        num_cores=_NC, num_subcores=_NS)

    @functools.partial(
        pl.kernel, mesh=mesh,
        out_type=jax.ShapeDtypeStruct((m, d), jnp.float32),
        scratch_types=[
            pltpu.VMEM((chunks, c), jnp.int32),
            pltpu.VMEM((c, d), jnp.float32),
            pltpu.SemaphoreType.DMA,
        ],
    )
    def gather_k(me_hbm, flat_hbm, out_hbm, idx_v, rows_v, sem):
        wid = lax.axis_index("s") * _NC + lax.axis_index("c")
        pltpu.sync_copy(me_hbm.at[wid], idx_v)
        base = wid * (chunks * c)
        for j in range(chunks):
            pltpu.async_copy(flat_hbm.at[idx_v.at[j]], rows_v, sem).wait()
            pltpu.sync_copy(rows_v, out_hbm.at[pl.ds(base + j * c, c)])

    return gather_k(me3, flat)


def _tc_mixer(gathered, w1b, w2c, bm=1024):
    m, d = gathered.shape
    _, h = w1b.shape

    def body(x_ref, w1_ref, w2_ref, o_ref):
        xb = x_ref[...].astype(jnp.bfloat16)
        acts = jnp.dot(xb, w1_ref[...], preferred_element_type=jnp.float32)
        acts = jax.nn.gelu(acts)
        o_ref[...] = jnp.dot(acts, w2_ref[...],
                             preferred_element_type=jnp.float32)

    return pl.pallas_call(
        body,
        grid=(m // bm,),
        in_specs=[
            pl.BlockSpec((bm, d), lambda i: (i, 0)),
            pl.BlockSpec((d, h), lambda i: (0, 0)),
            pl.BlockSpec((h, 1), lambda i: (0, 0)),
        ],
        out_specs=pl.BlockSpec((bm, 1), lambda i: (i, 0)),
        out_shape=jax.ShapeDtypeStruct((m, 1), jnp.float32),
    )(gathered, w1b, w2c)


def _sc_scatter(preds3, me3, cu_pad, b, max_seqlen):
    ns, rows, c = me3.shape
    out_len = b * max_seqlen
    per_tile = out_len // ns
    nb = cu_pad.shape[0]

    mesh = plsc.VectorSubcoreMesh(
        core_axis_name="c", subcore_axis_name="s",
        num_cores=1, num_subcores=_NS)

    @functools.partial(
        pl.kernel, mesh=mesh,
        out_type=jax.ShapeDtypeStruct((out_len,), jnp.float32),
        scratch_types=[
            pltpu.VMEM((nb,), jnp.int32),
            pltpu.VMEM((rows, c), jnp.float32),
            pltpu.VMEM((rows, c), jnp.int32),
            pltpu.VMEM((rows, c), jnp.int32),
            pltpu.VMEM((per_tile,), jnp.float32),
            pltpu.SemaphoreType.DMA,
        ],
    )
    def scatter_k(preds_hbm, me_hbm, cu_hbm, out_hbm,
                  cu_v, pred_v, me_v, oidx_v, zbuf, sem):
        sid = lax.axis_index("s")
        pltpu.sync_copy(cu_hbm, cu_v)
        pltpu.sync_copy(preds_hbm.at[sid], pred_v)
        pltpu.sync_copy(me_hbm.at[sid], me_v)

        def zero_body(i, _):
            zbuf[pl.ds(i * _L, _L)] = jnp.zeros((_L,), jnp.float32)
            return 0
        lax.fori_loop(0, per_tile // _L, zero_body, 0)
        pltpu.sync_copy(zbuf, out_hbm.at[pl.ds(sid * per_tile, per_tile)])
        plsc.subcore_barrier()

        cuvec = cu_v[pl.ds(0, _L)]
        cu_bcast = [
            cuvec.at[jnp.full((_L,), j, jnp.int32)].get(
                mode="promise_in_bounds")
            for j in range(1, b)
        ]
        for r in range(rows):
            for k in range(c // _L):
                me16 = me_v[r, pl.ds(k * _L, _L)]
                doc16 = jnp.zeros((_L,), jnp.int32)
                for cu_j in cu_bcast:
                    doc16 = doc16 + jnp.where(me16 >= cu_j, 1, 0)
                base16 = cuvec.at[doc16].get(mode="promise_in_bounds")
                oidx_v[r, pl.ds(k * _L, _L)] = (
                    doc16 * max_seqlen + me16 - base16)

        for r in range(rows):
            pltpu.async_copy(pred_v.at[r], out_hbm.at[oidx_v.at[r]],
                             sem).wait()

    return scatter_k(preds3, me3, cu_pad)


def kernel(flat, cu_seqlens, match_ends, W1, w2):
    total_tok, d = flat.shape
    (m,) = match_ends.shape
    b = cu_seqlens.shape[0] - 1
    max_seqlen = 4096

    n_pipe = 4
    mc = m // n_pipe
    chunk = 32
    w1b = W1.astype(jnp.bfloat16)
    w2c = w2.reshape(d, 1).astype(jnp.float32)
    pred_chunks = []
    for i in range(n_pipe):
        me_i = lax.slice_in_dim(match_ends, i * mc, (i + 1) * mc)
        me3_i = me_i.reshape(_NW, mc // (_NW * chunk), chunk)
        g_i = _sc_gather(me3_i, flat)
        pred_chunks.append(_tc_mixer(g_i, w1b, w2c, bm=1024))
    preds = jnp.concatenate(pred_chunks, axis=0)

    cu_pad = jnp.concatenate(
        [cu_seqlens.astype(jnp.int32),
         jnp.zeros((32 - cu_seqlens.shape[0],), jnp.int32)])
    preds3 = preds.reshape(_NS, m // (_NS * 128), 128)
    me_sc = match_ends.reshape(_NS, m // (_NS * 128), 128)
    out_flat = _sc_scatter(preds3, me_sc, cu_pad, b, max_seqlen)
    return out_flat.reshape(b, max_seqlen)
